# Initial kernel scaffold; baseline (speedup 1.0000x reference)
import jax
import jax.numpy as jnp
from jax.experimental import pallas as pl


def kernel(f_atoms, f_bonds, a2b, b2a, b2revb, a_scope, b_scope, a2a, features_batch, W_i, W_h, W_o):
    raise NotImplementedError("write your pallas kernel here")


# speedup vs baseline: 1.2053x; 1.2053x over previous
import functools

import jax
import jax.numpy as jnp
from jax import lax
from jax.experimental import pallas as pl
from jax.experimental.pallas import tpu as pltpu
from jax.experimental.pallas import tpu_sc as plsc

N_ATOMS = 10000
N_BONDS = 320000
MAX_NB = 32
D = 128
L = 16
NL = D // L

NC, NS = 2, 16
NW = NC * NS
BPW = N_BONDS // NW
BCH = 80
NBCH = BPW // BCH
APW = 320
N_ATOMS_PAD = APW * NW
ACH = 4
NACH = APW // ACH


def _mesh():
    return plsc.VectorSubcoreMesh(core_axis_name="c", subcore_axis_name="s",
                                  num_cores=NC, num_subcores=NS)


def _wid():
    return lax.axis_index("s") * NC + lax.axis_index("c")


def _mm(x, w, relu, bm):
    m, k = x.shape
    n = w.shape[1]

    def body(x_ref, w_ref, o_ref):
        acc = jnp.dot(x_ref[...], w_ref[...], preferred_element_type=jnp.float32)
        if relu:
            acc = jnp.maximum(acc, 0.0)
        o_ref[...] = acc

    return pl.pallas_call(
        body,
        grid=(m // bm,),
        in_specs=[pl.BlockSpec((bm, k), lambda i: (i, 0)),
                  pl.BlockSpec((k, n), lambda i: (0, 0))],
        out_specs=pl.BlockSpec((bm, n), lambda i: (i, 0)),
        out_shape=jax.ShapeDtypeStruct((m, n), jnp.float32),
    )(x, w)


@functools.partial(
    pl.kernel,
    out_type=jax.ShapeDtypeStruct((N_BONDS, D), jnp.float32),
    mesh=_mesh(),
    scratch_types=[
        pltpu.VMEM((BPW,), jnp.int32),
        pltpu.VMEM((BCH, D), jnp.float32),
        pltpu.VMEM((BCH, D), jnp.float32),
        pltpu.SemaphoreType.DMA,
    ],
)
def _inp_kernel(faproj, fbproj, b2a, out, idx_v, ga_v, fb_v, sem):
    base = pl.multiple_of(_wid() * BPW, 8)
    pltpu.sync_copy(b2a.at[pl.ds(base, BPW)], idx_v)

    def chunk(g, _):
        off = pl.multiple_of(g * BCH, 8)
        row0 = pl.multiple_of(base + off, 8)
        cp = pltpu.async_copy(faproj.at[idx_v.at[pl.ds(off, BCH)]], ga_v, sem)
        pltpu.sync_copy(fbproj.at[pl.ds(row0, BCH)], fb_v)
        cp.wait()

        def rows(r, _):
            for c in range(NL):
                v = ga_v[r, pl.ds(c * L, L)] + fb_v[r, pl.ds(c * L, L)]
                fb_v[r, pl.ds(c * L, L)] = jnp.maximum(v, 0.0)
            return 0

        lax.fori_loop(0, BCH, rows, 0)
        pltpu.sync_copy(fb_v, out.at[pl.ds(row0, BCH)])
        return 0

    lax.fori_loop(0, NBCH, chunk, 0)


@functools.partial(
    pl.kernel,
    out_type=jax.ShapeDtypeStruct((N_ATOMS_PAD, D), jnp.float32),
    mesh=_mesh(),
    scratch_types=[
        pltpu.VMEM((APW * MAX_NB,), jnp.int32),
        pltpu.VMEM((ACH * MAX_NB, D), jnp.float32),
        pltpu.VMEM((APW, D), jnp.float32),
        pltpu.SemaphoreType.DMA,
    ],
)
def _segsum_kernel(h, a2b_flat, out, idx_v, rows_v, out_v, sem):
    abase = pl.multiple_of(_wid() * APW, 8)
    pltpu.sync_copy(a2b_flat.at[pl.ds(abase * MAX_NB, APW * MAX_NB)], idx_v)

    def chunk(g, _):
        off = pl.multiple_of(g * (ACH * MAX_NB), 8)
        pltpu.async_copy(h.at[idx_v.at[pl.ds(off, ACH * MAX_NB)]], rows_v, sem).wait()
        for a in range(ACH):
            for c in range(NL):
                def red(r, acc):
                    return acc + rows_v[a * MAX_NB + r, pl.ds(c * L, L)]
                s = lax.fori_loop(0, MAX_NB, red, jnp.zeros((L,), jnp.float32))
                out_v[g * ACH + a, pl.ds(c * L, L)] = s
        return 0

    lax.fori_loop(0, NACH, chunk, 0)
    pltpu.sync_copy(out_v, out.at[pl.ds(abase, APW)])


@functools.partial(
    pl.kernel,
    out_type=jax.ShapeDtypeStruct((N_BONDS, D), jnp.float32),
    mesh=_mesh(),
    scratch_types=[
        pltpu.VMEM((BPW,), jnp.int32),
        pltpu.VMEM((BPW,), jnp.int32),
        pltpu.VMEM((BCH, D), jnp.float32),
        pltpu.VMEM((BCH, D), jnp.float32),
        pltpu.VMEM((BCH, D), jnp.float32),
        pltpu.SemaphoreType.DMA,
    ],
)
def _edge_update_kernel(am, hm, inp, b2a, b2revb, out,
                        ia_v, ib_v, ga_v, gb_v, pi_v, sem):
    base = pl.multiple_of(_wid() * BPW, 8)
    pltpu.sync_copy(b2a.at[pl.ds(base, BPW)], ia_v)
    pltpu.sync_copy(b2revb.at[pl.ds(base, BPW)], ib_v)

    def chunk(g, _):
        off = pl.multiple_of(g * BCH, 8)
        row0 = pl.multiple_of(base + off, 8)
        c1 = pltpu.async_copy(am.at[ia_v.at[pl.ds(off, BCH)]], ga_v, sem)
        c2 = pltpu.async_copy(hm.at[ib_v.at[pl.ds(off, BCH)]], gb_v, sem)
        pltpu.sync_copy(inp.at[pl.ds(row0, BCH)], pi_v)
        c1.wait()
        c2.wait()

        def rows(r, _):
            for c in range(NL):
                v = pi_v[r, pl.ds(c * L, L)] + ga_v[r, pl.ds(c * L, L)] \
                    - gb_v[r, pl.ds(c * L, L)]
                pi_v[r, pl.ds(c * L, L)] = jnp.maximum(v, 0.0)
            return 0

        lax.fori_loop(0, BCH, rows, 0)
        pltpu.sync_copy(pi_v, out.at[pl.ds(row0, BCH)])
        return 0

    lax.fori_loop(0, NBCH, chunk, 0)


def kernel(f_atoms, f_bonds, a2b, b2a, b2revb, a_scope, b_scope, a2a,
           features_batch, W_i, W_h, W_o):
    b2a = b2a.astype(jnp.int32)
    b2revb = b2revb.astype(jnp.int32)
    a2b = a2b.astype(jnp.int32)
    a2b_flat = jnp.pad(a2b, ((0, N_ATOMS_PAD - N_ATOMS), (0, 0))).reshape(-1)

    fa_proj = _mm(f_atoms, W_i[:D], relu=False, bm=400)
    fb_proj = _mm(f_bonds, W_i[D:], relu=False, bm=2000)
    inp = _inp_kernel(fa_proj, fb_proj, b2a)

    h = inp
    for i in range(W_h.shape[0]):
        a_msg = _segsum_kernel(h, a2b_flat)
        am = _mm(a_msg, W_h[i], relu=False, bm=512)
        hm = _mm(h, W_h[i], relu=False, bm=2000)
        h = _edge_update_kernel(am, hm, inp, b2a, b2revb)

    return _mm(h, W_o, relu=True, bm=2000)


# speedup vs baseline: 1.6591x; 1.3766x over previous
import functools

import jax
import jax.numpy as jnp
from jax import lax
from jax.experimental import pallas as pl
from jax.experimental.pallas import tpu as pltpu
from jax.experimental.pallas import tpu_sc as plsc

N_ATOMS = 10000
N_BONDS = 320000
MAX_NB = 32
D = 128
L = 16
NL = D // L

NC, NS = 2, 16
NW = NC * NS
BPW = N_BONDS // NW
BCH = 80
NBCH = BPW // BCH
APW = 320
N_ATOMS_PAD = APW * NW
ACH = 4
NACH = APW // ACH


def _mesh():
    return plsc.VectorSubcoreMesh(core_axis_name="c", subcore_axis_name="s",
                                  num_cores=NC, num_subcores=NS)


def _wid():
    return lax.axis_index("s") * NC + lax.axis_index("c")


def _mm(x, w, relu, bm):
    m, k = x.shape
    n = w.shape[1]

    def body(x_ref, w_ref, o_ref):
        acc = jnp.dot(x_ref[...], w_ref[...], preferred_element_type=jnp.float32)
        if relu:
            acc = jnp.maximum(acc, 0.0)
        o_ref[...] = acc

    return pl.pallas_call(
        body,
        grid=(m // bm,),
        in_specs=[pl.BlockSpec((bm, k), lambda i: (i, 0)),
                  pl.BlockSpec((k, n), lambda i: (0, 0))],
        out_specs=pl.BlockSpec((bm, n), lambda i: (i, 0)),
        out_shape=jax.ShapeDtypeStruct((m, n), jnp.float32),
    )(x, w)


@functools.partial(
    pl.kernel,
    out_type=jax.ShapeDtypeStruct((N_BONDS, D), jnp.float32),
    mesh=_mesh(),
    scratch_types=[
        pltpu.VMEM((BPW,), jnp.int32),
        pltpu.VMEM((BCH, D), jnp.float32), pltpu.VMEM((BCH, D), jnp.float32),
        pltpu.VMEM((BCH, D), jnp.float32), pltpu.VMEM((BCH, D), jnp.float32),
        pltpu.VMEM((BCH, D), jnp.float32), pltpu.VMEM((BCH, D), jnp.float32),
        pltpu.SemaphoreType.DMA, pltpu.SemaphoreType.DMA,
        pltpu.SemaphoreType.DMA, pltpu.SemaphoreType.DMA,
    ],
)
def _inp_kernel(faproj, fbproj, b2a, out,
                idx_v, ga0, ga1, fi0, fi1, po0, po1, ls0, ls1, ws0, ws1):
    ga, fi, po = (ga0, ga1), (fi0, fi1), (po0, po1)
    ls, ws = (ls0, ls1), (ws0, ws1)
    base = pl.multiple_of(_wid() * BPW, 8)
    pltpu.sync_copy(b2a.at[pl.ds(base, BPW)], idx_v)

    def issue(g, s):
        off = pl.multiple_of(g * BCH, 8)
        row0 = pl.multiple_of(base + off, 8)
        pltpu.async_copy(faproj.at[idx_v.at[pl.ds(off, BCH)]], ga[s], ls[s])
        pltpu.async_copy(fbproj.at[pl.ds(row0, BCH)], fi[s], ls[s])

    issue(0, 0)
    issue(1, 1)

    def outer(go, _):
        for s in range(2):
            g = 2 * go + s

            @pl.when(g < NBCH)
            def _():
                pltpu.make_async_copy(faproj.at[idx_v.at[pl.ds(0, BCH)]],
                                      ga[s], ls[s]).wait()
                pltpu.make_async_copy(fbproj.at[pl.ds(0, BCH)], fi[s], ls[s]).wait()

                @pl.when(go > 0)
                def _():
                    pltpu.make_async_copy(po[s], out.at[pl.ds(0, BCH)], ws[s]).wait()

                def rows(r, _):
                    for c in range(NL):
                        v = ga[s][r, pl.ds(c * L, L)] + fi[s][r, pl.ds(c * L, L)]
                        po[s][r, pl.ds(c * L, L)] = jnp.maximum(v, 0.0)
                    return 0

                lax.fori_loop(0, BCH, rows, 0)
                row0 = pl.multiple_of(base + pl.multiple_of(g * BCH, 8), 8)
                pltpu.async_copy(po[s], out.at[pl.ds(row0, BCH)], ws[s])

                @pl.when(g + 2 < NBCH)
                def _():
                    issue(g + 2, s)
        return 0

    lax.fori_loop(0, (NBCH + 1) // 2, outer, 0)
    pltpu.make_async_copy(po[0], out.at[pl.ds(0, BCH)], ws[0]).wait()
    pltpu.make_async_copy(po[1], out.at[pl.ds(0, BCH)], ws[1]).wait()


@functools.partial(
    pl.kernel,
    out_type=jax.ShapeDtypeStruct((N_ATOMS_PAD, D), jnp.float32),
    mesh=_mesh(),
    scratch_types=[
        pltpu.VMEM((APW * MAX_NB,), jnp.int32),
        pltpu.VMEM((ACH * MAX_NB, D), jnp.float32),
        pltpu.VMEM((ACH * MAX_NB, D), jnp.float32),
        pltpu.VMEM((APW, D), jnp.float32),
        pltpu.SemaphoreType.DMA, pltpu.SemaphoreType.DMA,
    ],
)
def _segsum_kernel(h, a2b_flat, out, idx_v, r0, r1, out_v, s0, s1):
    rows, sems = (r0, r1), (s0, s1)
    abase = pl.multiple_of(_wid() * APW, 8)
    pltpu.sync_copy(a2b_flat.at[pl.ds(abase * MAX_NB, APW * MAX_NB)], idx_v)

    def issue(g, s):
        off = pl.multiple_of(g * (ACH * MAX_NB), 8)
        pltpu.async_copy(h.at[idx_v.at[pl.ds(off, ACH * MAX_NB)]], rows[s], sems[s])

    issue(0, 0)
    issue(1, 1)

    def outer(go, _):
        for s in range(2):
            g = 2 * go + s
            pltpu.make_async_copy(h.at[idx_v.at[pl.ds(0, ACH * MAX_NB)]],
                                  rows[s], sems[s]).wait()
            for a in range(ACH):
                for c in range(NL):
                    def red(r, acc):
                        return acc + rows[s][a * MAX_NB + r, pl.ds(c * L, L)]
                    out_v[g * ACH + a, pl.ds(c * L, L)] = lax.fori_loop(
                        0, MAX_NB, red, jnp.zeros((L,), jnp.float32))

            @pl.when(g + 2 < NACH)
            def _():
                issue(g + 2, s)
        return 0

    lax.fori_loop(0, NACH // 2, outer, 0)
    pltpu.sync_copy(out_v, out.at[pl.ds(abase, APW)])


@functools.partial(
    pl.kernel,
    out_type=jax.ShapeDtypeStruct((N_BONDS, D), jnp.float32),
    mesh=_mesh(),
    scratch_types=[
        pltpu.VMEM((BPW,), jnp.int32), pltpu.VMEM((BPW,), jnp.int32),
        pltpu.VMEM((BCH, D), jnp.float32), pltpu.VMEM((BCH, D), jnp.float32),
        pltpu.VMEM((BCH, D), jnp.float32), pltpu.VMEM((BCH, D), jnp.float32),
        pltpu.VMEM((BCH, D), jnp.float32), pltpu.VMEM((BCH, D), jnp.float32),
        pltpu.VMEM((BCH, D), jnp.float32), pltpu.VMEM((BCH, D), jnp.float32),
        pltpu.SemaphoreType.DMA, pltpu.SemaphoreType.DMA,
        pltpu.SemaphoreType.DMA, pltpu.SemaphoreType.DMA,
    ],
)
def _edge_update_kernel(am, hm, inp, b2a, b2revb, out, ia_v, ib_v,
                        ga0, ga1, gb0, gb1, pi0, pi1, po0, po1,
                        ls0, ls1, ws0, ws1):
    ga, gb, pi, po = (ga0, ga1), (gb0, gb1), (pi0, pi1), (po0, po1)
    ls, ws = (ls0, ls1), (ws0, ws1)
    base = pl.multiple_of(_wid() * BPW, 8)
    pltpu.sync_copy(b2a.at[pl.ds(base, BPW)], ia_v)
    pltpu.sync_copy(b2revb.at[pl.ds(base, BPW)], ib_v)

    def issue(g, s):
        off = pl.multiple_of(g * BCH, 8)
        row0 = pl.multiple_of(base + off, 8)
        pltpu.async_copy(am.at[ia_v.at[pl.ds(off, BCH)]], ga[s], ls[s])
        pltpu.async_copy(hm.at[ib_v.at[pl.ds(off, BCH)]], gb[s], ls[s])
        pltpu.async_copy(inp.at[pl.ds(row0, BCH)], pi[s], ls[s])

    issue(0, 0)
    issue(1, 1)

    def outer(go, _):
        for s in range(2):
            g = 2 * go + s

            @pl.when(g < NBCH)
            def _():
                pltpu.make_async_copy(am.at[ia_v.at[pl.ds(0, BCH)]],
                                      ga[s], ls[s]).wait()
                pltpu.make_async_copy(hm.at[ib_v.at[pl.ds(0, BCH)]],
                                      gb[s], ls[s]).wait()
                pltpu.make_async_copy(inp.at[pl.ds(0, BCH)], pi[s], ls[s]).wait()

                @pl.when(go > 0)
                def _():
                    pltpu.make_async_copy(po[s], out.at[pl.ds(0, BCH)], ws[s]).wait()

                def rows(r, _):
                    for c in range(NL):
                        v = pi[s][r, pl.ds(c * L, L)] + ga[s][r, pl.ds(c * L, L)] \
                            - gb[s][r, pl.ds(c * L, L)]
                        po[s][r, pl.ds(c * L, L)] = jnp.maximum(v, 0.0)
                    return 0

                lax.fori_loop(0, BCH, rows, 0)
                row0 = pl.multiple_of(base + pl.multiple_of(g * BCH, 8), 8)
                pltpu.async_copy(po[s], out.at[pl.ds(row0, BCH)], ws[s])

                @pl.when(g + 2 < NBCH)
                def _():
                    issue(g + 2, s)
        return 0

    lax.fori_loop(0, (NBCH + 1) // 2, outer, 0)
    pltpu.make_async_copy(po[0], out.at[pl.ds(0, BCH)], ws[0]).wait()
    pltpu.make_async_copy(po[1], out.at[pl.ds(0, BCH)], ws[1]).wait()


def kernel(f_atoms, f_bonds, a2b, b2a, b2revb, a_scope, b_scope, a2a,
           features_batch, W_i, W_h, W_o):
    b2a = b2a.astype(jnp.int32)
    b2revb = b2revb.astype(jnp.int32)
    a2b = a2b.astype(jnp.int32)
    a2b_flat = jnp.pad(a2b, ((0, N_ATOMS_PAD - N_ATOMS), (0, 0))).reshape(-1)

    fa_proj = _mm(f_atoms, W_i[:D], relu=False, bm=400)
    fb_proj = _mm(f_bonds, W_i[D:], relu=False, bm=2000)
    inp = _inp_kernel(fa_proj, fb_proj, b2a)

    h = inp
    for i in range(W_h.shape[0]):
        a_msg = _segsum_kernel(h, a2b_flat)
        am = _mm(a_msg, W_h[i], relu=False, bm=512)
        hm = _mm(h, W_h[i], relu=False, bm=2000)
        h = _edge_update_kernel(am, hm, inp, b2a, b2revb)

    return _mm(h, W_o, relu=True, bm=2000)


# speedup vs baseline: 1.6610x; 1.0011x over previous
import functools

import jax
import jax.numpy as jnp
from jax import lax
from jax.experimental import pallas as pl
from jax.experimental.pallas import tpu as pltpu
from jax.experimental.pallas import tpu_sc as plsc

N_ATOMS = 10000
N_BONDS = 320000
MAX_NB = 32
D = 128
L = 16
NL = D // L

NC, NS = 2, 16
NW = NC * NS
BPW = N_BONDS // NW
BCH = 80
NBCH = BPW // BCH
APW = 320
N_ATOMS_PAD = APW * NW
ACH = 4
NACH = APW // ACH


def _mesh():
    return plsc.VectorSubcoreMesh(core_axis_name="c", subcore_axis_name="s",
                                  num_cores=NC, num_subcores=NS)


def _wid():
    return lax.axis_index("s") * NC + lax.axis_index("c")


def _mm(x, w, relu, bm):
    m, k = x.shape
    n = w.shape[1]

    def body(x_ref, w_ref, o_ref):
        acc = jnp.dot(x_ref[...], w_ref[...], preferred_element_type=jnp.float32)
        if relu:
            acc = jnp.maximum(acc, 0.0)
        o_ref[...] = acc

    return pl.pallas_call(
        body,
        grid=(m // bm,),
        in_specs=[pl.BlockSpec((bm, k), lambda i: (i, 0)),
                  pl.BlockSpec((k, n), lambda i: (0, 0))],
        out_specs=pl.BlockSpec((bm, n), lambda i: (i, 0)),
        out_shape=jax.ShapeDtypeStruct((m, n), jnp.float32),
    )(x, w)


@functools.partial(
    pl.kernel,
    out_type=jax.ShapeDtypeStruct((N_BONDS, D), jnp.float32),
    mesh=_mesh(),
    scratch_types=[
        pltpu.VMEM((BPW,), jnp.int32),
        pltpu.VMEM((BCH, D), jnp.float32), pltpu.VMEM((BCH, D), jnp.float32),
        pltpu.VMEM((BCH, D), jnp.float32), pltpu.VMEM((BCH, D), jnp.float32),
        pltpu.VMEM((BCH, D), jnp.float32), pltpu.VMEM((BCH, D), jnp.float32),
        pltpu.SemaphoreType.DMA, pltpu.SemaphoreType.DMA,
        pltpu.SemaphoreType.DMA, pltpu.SemaphoreType.DMA,
    ],
)
def _inp_kernel(faproj, fbproj, b2a, out,
                idx_v, ga0, ga1, fi0, fi1, po0, po1, ls0, ls1, ws0, ws1):
    ga, fi, po = (ga0, ga1), (fi0, fi1), (po0, po1)
    ls, ws = (ls0, ls1), (ws0, ws1)
    base = pl.multiple_of(_wid() * BPW, 8)
    pltpu.sync_copy(b2a.at[pl.ds(base, BPW)], idx_v)

    def issue(g, s):
        off = pl.multiple_of(g * BCH, 8)
        row0 = pl.multiple_of(base + off, 8)
        pltpu.async_copy(faproj.at[idx_v.at[pl.ds(off, BCH)]], ga[s], ls[s])
        pltpu.async_copy(fbproj.at[pl.ds(row0, BCH)], fi[s], ls[s])

    issue(0, 0)
    issue(1, 1)

    def outer(go, _):
        for s in range(2):
            g = 2 * go + s

            @pl.when(g < NBCH)
            def _():
                pltpu.make_async_copy(faproj.at[idx_v.at[pl.ds(0, BCH)]],
                                      ga[s], ls[s]).wait()
                pltpu.make_async_copy(fbproj.at[pl.ds(0, BCH)], fi[s], ls[s]).wait()

                @pl.when(go > 0)
                def _():
                    pltpu.make_async_copy(po[s], out.at[pl.ds(0, BCH)], ws[s]).wait()

                def rows(r, _):
                    for c in range(NL):
                        v = ga[s][r, pl.ds(c * L, L)] + fi[s][r, pl.ds(c * L, L)]
                        po[s][r, pl.ds(c * L, L)] = jnp.maximum(v, 0.0)
                    return 0

                lax.fori_loop(0, BCH, rows, 0)
                row0 = pl.multiple_of(base + pl.multiple_of(g * BCH, 8), 8)
                pltpu.async_copy(po[s], out.at[pl.ds(row0, BCH)], ws[s])

                @pl.when(g + 2 < NBCH)
                def _():
                    issue(g + 2, s)
        return 0

    lax.fori_loop(0, (NBCH + 1) // 2, outer, 0)
    pltpu.make_async_copy(po[0], out.at[pl.ds(0, BCH)], ws[0]).wait()
    pltpu.make_async_copy(po[1], out.at[pl.ds(0, BCH)], ws[1]).wait()


SEG_SLOTS = 4

SEG_ROWS = ACH * MAX_NB

@functools.partial(
    pl.kernel,
    out_type=jax.ShapeDtypeStruct((N_ATOMS_PAD, D), jnp.float32),
    mesh=_mesh(),
    scratch_types=[
        pltpu.VMEM((APW * MAX_NB,), jnp.int32),
        pltpu.VMEM((SEG_SLOTS * SEG_ROWS, D), jnp.float32),
        pltpu.VMEM((APW, D), jnp.float32),
        pltpu.SemaphoreType.DMA, pltpu.SemaphoreType.DMA,
        pltpu.SemaphoreType.DMA, pltpu.SemaphoreType.DMA,
    ],
)
def _segsum_kernel(h, a2b_flat, out, idx_v, rows_v, out_v, s0, s1, s2, s3):
    sems = (s0, s1, s2, s3)
    abase = pl.multiple_of(_wid() * APW, 8)
    pltpu.sync_copy(a2b_flat.at[pl.ds(abase * MAX_NB, APW * MAX_NB)], idx_v)

    def issue(g, s):
        off = pl.multiple_of(g * SEG_ROWS, 8)
        pltpu.async_copy(h.at[idx_v.at[pl.ds(off, SEG_ROWS)]],
                         rows_v.at[pl.ds(s * SEG_ROWS, SEG_ROWS)], sems[s])

    def wait(s):
        pltpu.make_async_copy(h.at[idx_v.at[pl.ds(0, SEG_ROWS)]],
                              rows_v.at[pl.ds(s * SEG_ROWS, SEG_ROWS)],
                              sems[s]).wait()

    for s in range(SEG_SLOTS):
        issue(s, s)

    def chunk(g, _):
        s = lax.rem(g, SEG_SLOTS)
        lax.switch(s, [lambda k=k: wait(k) for k in range(SEG_SLOTS)])
        s_off = s * SEG_ROWS

        def do_atom(a, _):
            for c in range(NL):
                def red(r, acc):
                    return acc + rows_v[s_off + a * MAX_NB + r, pl.ds(c * L, L)]
                out_v[g * ACH + a, pl.ds(c * L, L)] = lax.fori_loop(
                    0, MAX_NB, red, jnp.zeros((L,), jnp.float32))
            return 0

        lax.fori_loop(0, ACH, do_atom, 0)

        @pl.when(g + SEG_SLOTS < NACH)
        def _():
            lax.switch(s, [lambda k=k: issue(g + SEG_SLOTS, k)
                           for k in range(SEG_SLOTS)])
        return 0

    lax.fori_loop(0, NACH, chunk, 0)
    pltpu.sync_copy(out_v, out.at[pl.ds(abase, APW)])


@functools.partial(
    pl.kernel,
    out_type=jax.ShapeDtypeStruct((N_BONDS, D), jnp.float32),
    mesh=_mesh(),
    scratch_types=[
        pltpu.VMEM((BPW,), jnp.int32), pltpu.VMEM((BPW,), jnp.int32),
        pltpu.VMEM((BCH, D), jnp.float32), pltpu.VMEM((BCH, D), jnp.float32),
        pltpu.VMEM((BCH, D), jnp.float32), pltpu.VMEM((BCH, D), jnp.float32),
        pltpu.VMEM((BCH, D), jnp.float32), pltpu.VMEM((BCH, D), jnp.float32),
        pltpu.VMEM((BCH, D), jnp.float32), pltpu.VMEM((BCH, D), jnp.float32),
        pltpu.SemaphoreType.DMA, pltpu.SemaphoreType.DMA,
        pltpu.SemaphoreType.DMA, pltpu.SemaphoreType.DMA,
    ],
)
def _edge_update_kernel(am, hm, inp, b2a, b2revb, out, ia_v, ib_v,
                        ga0, ga1, gb0, gb1, pi0, pi1, po0, po1,
                        ls0, ls1, ws0, ws1):
    ga, gb, pi, po = (ga0, ga1), (gb0, gb1), (pi0, pi1), (po0, po1)
    ls, ws = (ls0, ls1), (ws0, ws1)
    base = pl.multiple_of(_wid() * BPW, 8)
    pltpu.sync_copy(b2a.at[pl.ds(base, BPW)], ia_v)
    pltpu.sync_copy(b2revb.at[pl.ds(base, BPW)], ib_v)

    def issue(g, s):
        off = pl.multiple_of(g * BCH, 8)
        row0 = pl.multiple_of(base + off, 8)
        pltpu.async_copy(am.at[ia_v.at[pl.ds(off, BCH)]], ga[s], ls[s])
        pltpu.async_copy(hm.at[ib_v.at[pl.ds(off, BCH)]], gb[s], ls[s])
        pltpu.async_copy(inp.at[pl.ds(row0, BCH)], pi[s], ls[s])

    issue(0, 0)
    issue(1, 1)

    def outer(go, _):
        for s in range(2):
            g = 2 * go + s

            @pl.when(g < NBCH)
            def _():
                pltpu.make_async_copy(am.at[ia_v.at[pl.ds(0, BCH)]],
                                      ga[s], ls[s]).wait()
                pltpu.make_async_copy(hm.at[ib_v.at[pl.ds(0, BCH)]],
                                      gb[s], ls[s]).wait()
                pltpu.make_async_copy(inp.at[pl.ds(0, BCH)], pi[s], ls[s]).wait()

                @pl.when(go > 0)
                def _():
                    pltpu.make_async_copy(po[s], out.at[pl.ds(0, BCH)], ws[s]).wait()

                def rows(r, _):
                    for c in range(NL):
                        v = pi[s][r, pl.ds(c * L, L)] + ga[s][r, pl.ds(c * L, L)] \
                            - gb[s][r, pl.ds(c * L, L)]
                        po[s][r, pl.ds(c * L, L)] = jnp.maximum(v, 0.0)
                    return 0

                lax.fori_loop(0, BCH, rows, 0)
                row0 = pl.multiple_of(base + pl.multiple_of(g * BCH, 8), 8)
                pltpu.async_copy(po[s], out.at[pl.ds(row0, BCH)], ws[s])

                @pl.when(g + 2 < NBCH)
                def _():
                    issue(g + 2, s)
        return 0

    lax.fori_loop(0, (NBCH + 1) // 2, outer, 0)
    pltpu.make_async_copy(po[0], out.at[pl.ds(0, BCH)], ws[0]).wait()
    pltpu.make_async_copy(po[1], out.at[pl.ds(0, BCH)], ws[1]).wait()


def kernel(f_atoms, f_bonds, a2b, b2a, b2revb, a_scope, b_scope, a2a,
           features_batch, W_i, W_h, W_o):
    b2a = b2a.astype(jnp.int32)
    b2revb = b2revb.astype(jnp.int32)
    a2b = a2b.astype(jnp.int32)
    a2b_flat = jnp.pad(a2b, ((0, N_ATOMS_PAD - N_ATOMS), (0, 0))).reshape(-1)

    fa_proj = _mm(f_atoms, W_i[:D], relu=False, bm=400)
    fb_proj = _mm(f_bonds, W_i[D:], relu=False, bm=2000)
    inp = _inp_kernel(fa_proj, fb_proj, b2a)

    h = inp
    for i in range(W_h.shape[0]):
        a_msg = _segsum_kernel(h, a2b_flat)
        am = _mm(a_msg, W_h[i], relu=False, bm=512)
        hm = _mm(h, W_h[i], relu=False, bm=2000)
        h = _edge_update_kernel(am, hm, inp, b2a, b2revb)

    return _mm(h, W_o, relu=True, bm=2000)


# speedup vs baseline: 1.7067x; 1.0275x over previous
import functools

import jax
import jax.numpy as jnp
from jax import lax
from jax.experimental import pallas as pl
from jax.experimental.pallas import tpu as pltpu
from jax.experimental.pallas import tpu_sc as plsc

N_ATOMS = 10000
N_BONDS = 320000
MAX_NB = 32
D = 128
L = 16
NL = D // L

NC, NS = 2, 16
NW = NC * NS
BPW = N_BONDS // NW
BCH = 80
NBCH = BPW // BCH
APW = 320
N_ATOMS_PAD = APW * NW
ACH = 4
NACH = APW // ACH


def _mesh():
    return plsc.VectorSubcoreMesh(core_axis_name="c", subcore_axis_name="s",
                                  num_cores=NC, num_subcores=NS)


def _wid():
    return lax.axis_index("s") * NC + lax.axis_index("c")


def _mm(x, w, relu, bm):
    m, k = x.shape
    n = w.shape[1]

    def body(x_ref, w_ref, o_ref):
        acc = jnp.dot(x_ref[...], w_ref[...], preferred_element_type=jnp.float32)
        if relu:
            acc = jnp.maximum(acc, 0.0)
        o_ref[...] = acc

    return pl.pallas_call(
        body,
        grid=(m // bm,),
        in_specs=[pl.BlockSpec((bm, k), lambda i: (i, 0)),
                  pl.BlockSpec((k, n), lambda i: (0, 0))],
        out_specs=pl.BlockSpec((bm, n), lambda i: (i, 0)),
        out_shape=jax.ShapeDtypeStruct((m, n), jnp.float32),
    )(x, w)


@functools.partial(
    pl.kernel,
    out_type=jax.ShapeDtypeStruct((N_BONDS, D), jnp.float32),
    mesh=_mesh(),
    scratch_types=[
        pltpu.VMEM((BPW,), jnp.int32),
        pltpu.VMEM((BCH, D), jnp.float32), pltpu.VMEM((BCH, D), jnp.float32),
        pltpu.VMEM((BCH, D), jnp.float32), pltpu.VMEM((BCH, D), jnp.float32),
        pltpu.VMEM((BCH, D), jnp.float32), pltpu.VMEM((BCH, D), jnp.float32),
        pltpu.SemaphoreType.DMA, pltpu.SemaphoreType.DMA,
        pltpu.SemaphoreType.DMA, pltpu.SemaphoreType.DMA,
    ],
)
def _inp_kernel(faproj, fbproj, b2a, out,
                idx_v, ga0, ga1, fi0, fi1, po0, po1, ls0, ls1, ws0, ws1):
    ga, fi, po = (ga0, ga1), (fi0, fi1), (po0, po1)
    ls, ws = (ls0, ls1), (ws0, ws1)
    base = pl.multiple_of(_wid() * BPW, 8)
    pltpu.sync_copy(b2a.at[pl.ds(base, BPW)], idx_v)

    def issue(g, s):
        off = pl.multiple_of(g * BCH, 8)
        row0 = pl.multiple_of(base + off, 8)
        pltpu.async_copy(faproj.at[idx_v.at[pl.ds(off, BCH)]], ga[s], ls[s])
        pltpu.async_copy(fbproj.at[pl.ds(row0, BCH)], fi[s], ls[s])

    issue(0, 0)
    issue(1, 1)

    def outer(go, _):
        for s in range(2):
            g = 2 * go + s

            @pl.when(g < NBCH)
            def _():
                pltpu.make_async_copy(faproj.at[idx_v.at[pl.ds(0, BCH)]],
                                      ga[s], ls[s]).wait()
                pltpu.make_async_copy(fbproj.at[pl.ds(0, BCH)], fi[s], ls[s]).wait()

                @pl.when(go > 0)
                def _():
                    pltpu.make_async_copy(po[s], out.at[pl.ds(0, BCH)], ws[s]).wait()

                def rows(r, _):
                    for c in range(NL):
                        v = ga[s][r, pl.ds(c * L, L)] + fi[s][r, pl.ds(c * L, L)]
                        po[s][r, pl.ds(c * L, L)] = jnp.maximum(v, 0.0)
                    return 0

                lax.fori_loop(0, BCH, rows, 0)
                row0 = pl.multiple_of(base + pl.multiple_of(g * BCH, 8), 8)
                pltpu.async_copy(po[s], out.at[pl.ds(row0, BCH)], ws[s])

                @pl.when(g + 2 < NBCH)
                def _():
                    issue(g + 2, s)
        return 0

    lax.fori_loop(0, (NBCH + 1) // 2, outer, 0)
    pltpu.make_async_copy(po[0], out.at[pl.ds(0, BCH)], ws[0]).wait()
    pltpu.make_async_copy(po[1], out.at[pl.ds(0, BCH)], ws[1]).wait()


SEG_SLOTS = 4

SEG_ROWS = ACH * MAX_NB

@functools.partial(
    pl.kernel,
    out_type=jax.ShapeDtypeStruct((N_ATOMS_PAD, D), jnp.float32),
    mesh=_mesh(),
    scratch_types=[
        pltpu.VMEM((APW * MAX_NB,), jnp.int32),
        pltpu.VMEM((SEG_SLOTS * SEG_ROWS, D), jnp.float32),
        pltpu.VMEM((APW, D), jnp.float32),
        pltpu.SemaphoreType.DMA, pltpu.SemaphoreType.DMA,
        pltpu.SemaphoreType.DMA, pltpu.SemaphoreType.DMA,
    ],
)
def _segsum_kernel(h, a2b_flat, out, idx_v, rows_v, out_v, s0, s1, s2, s3):
    sems = (s0, s1, s2, s3)
    abase = pl.multiple_of(_wid() * APW, 8)
    pltpu.sync_copy(a2b_flat.at[pl.ds(abase * MAX_NB, APW * MAX_NB)], idx_v)

    def issue(g, s):
        off = pl.multiple_of(g * SEG_ROWS, 8)
        pltpu.async_copy(h.at[idx_v.at[pl.ds(off, SEG_ROWS)]],
                         rows_v.at[pl.ds(s * SEG_ROWS, SEG_ROWS)], sems[s])

    def wait(s):
        pltpu.make_async_copy(h.at[idx_v.at[pl.ds(0, SEG_ROWS)]],
                              rows_v.at[pl.ds(s * SEG_ROWS, SEG_ROWS)],
                              sems[s]).wait()

    for s in range(SEG_SLOTS):
        issue(s, s)

    def chunk(g, _):
        s = lax.rem(g, SEG_SLOTS)
        lax.switch(s, [lambda k=k: wait(k) for k in range(SEG_SLOTS)])
        s_off = s * SEG_ROWS

        def do_atom(a, _):
            r0 = s_off + a * MAX_NB
            for c in range(NL):
                accs = [rows_v[r0 + r, pl.ds(c * L, L)] for r in range(4)]
                for r in range(4, MAX_NB):
                    accs[r % 4] = accs[r % 4] + rows_v[r0 + r, pl.ds(c * L, L)]
                out_v[g * ACH + a, pl.ds(c * L, L)] = (
                    (accs[0] + accs[1]) + (accs[2] + accs[3]))
            return 0

        lax.fori_loop(0, ACH, do_atom, 0)

        @pl.when(g + SEG_SLOTS < NACH)
        def _():
            lax.switch(s, [lambda k=k: issue(g + SEG_SLOTS, k)
                           for k in range(SEG_SLOTS)])
        return 0

    lax.fori_loop(0, NACH, chunk, 0)
    pltpu.sync_copy(out_v, out.at[pl.ds(abase, APW)])


@functools.partial(
    pl.kernel,
    out_type=jax.ShapeDtypeStruct((N_BONDS, D), jnp.float32),
    mesh=_mesh(),
    scratch_types=[
        pltpu.VMEM((BPW,), jnp.int32), pltpu.VMEM((BPW,), jnp.int32),
        pltpu.VMEM((BCH, D), jnp.float32), pltpu.VMEM((BCH, D), jnp.float32),
        pltpu.VMEM((BCH, D), jnp.float32), pltpu.VMEM((BCH, D), jnp.float32),
        pltpu.VMEM((BCH, D), jnp.float32), pltpu.VMEM((BCH, D), jnp.float32),
        pltpu.VMEM((BCH, D), jnp.float32), pltpu.VMEM((BCH, D), jnp.float32),
        pltpu.SemaphoreType.DMA, pltpu.SemaphoreType.DMA,
        pltpu.SemaphoreType.DMA, pltpu.SemaphoreType.DMA,
    ],
)
def _edge_update_kernel(am, hm, inp, b2a, b2revb, out, ia_v, ib_v,
                        ga0, ga1, gb0, gb1, pi0, pi1, po0, po1,
                        ls0, ls1, ws0, ws1):
    ga, gb, pi, po = (ga0, ga1), (gb0, gb1), (pi0, pi1), (po0, po1)
    ls, ws = (ls0, ls1), (ws0, ws1)
    base = pl.multiple_of(_wid() * BPW, 8)
    pltpu.sync_copy(b2a.at[pl.ds(base, BPW)], ia_v)
    pltpu.sync_copy(b2revb.at[pl.ds(base, BPW)], ib_v)

    def issue(g, s):
        off = pl.multiple_of(g * BCH, 8)
        row0 = pl.multiple_of(base + off, 8)
        pltpu.async_copy(am.at[ia_v.at[pl.ds(off, BCH)]], ga[s], ls[s])
        pltpu.async_copy(hm.at[ib_v.at[pl.ds(off, BCH)]], gb[s], ls[s])
        pltpu.async_copy(inp.at[pl.ds(row0, BCH)], pi[s], ls[s])

    issue(0, 0)
    issue(1, 1)

    def outer(go, _):
        for s in range(2):
            g = 2 * go + s

            @pl.when(g < NBCH)
            def _():
                pltpu.make_async_copy(am.at[ia_v.at[pl.ds(0, BCH)]],
                                      ga[s], ls[s]).wait()
                pltpu.make_async_copy(hm.at[ib_v.at[pl.ds(0, BCH)]],
                                      gb[s], ls[s]).wait()
                pltpu.make_async_copy(inp.at[pl.ds(0, BCH)], pi[s], ls[s]).wait()

                @pl.when(go > 0)
                def _():
                    pltpu.make_async_copy(po[s], out.at[pl.ds(0, BCH)], ws[s]).wait()

                def rows(r, _):
                    for c in range(NL):
                        v = pi[s][r, pl.ds(c * L, L)] + ga[s][r, pl.ds(c * L, L)] \
                            - gb[s][r, pl.ds(c * L, L)]
                        po[s][r, pl.ds(c * L, L)] = jnp.maximum(v, 0.0)
                    return 0

                lax.fori_loop(0, BCH, rows, 0)
                row0 = pl.multiple_of(base + pl.multiple_of(g * BCH, 8), 8)
                pltpu.async_copy(po[s], out.at[pl.ds(row0, BCH)], ws[s])

                @pl.when(g + 2 < NBCH)
                def _():
                    issue(g + 2, s)
        return 0

    lax.fori_loop(0, (NBCH + 1) // 2, outer, 0)
    pltpu.make_async_copy(po[0], out.at[pl.ds(0, BCH)], ws[0]).wait()
    pltpu.make_async_copy(po[1], out.at[pl.ds(0, BCH)], ws[1]).wait()


def kernel(f_atoms, f_bonds, a2b, b2a, b2revb, a_scope, b_scope, a2a,
           features_batch, W_i, W_h, W_o):
    b2a = b2a.astype(jnp.int32)
    b2revb = b2revb.astype(jnp.int32)
    a2b = a2b.astype(jnp.int32)
    a2b_flat = jnp.pad(a2b, ((0, N_ATOMS_PAD - N_ATOMS), (0, 0))).reshape(-1)

    fa_proj = _mm(f_atoms, W_i[:D], relu=False, bm=400)
    fb_proj = _mm(f_bonds, W_i[D:], relu=False, bm=2000)
    inp = _inp_kernel(fa_proj, fb_proj, b2a)

    h = inp
    for i in range(W_h.shape[0]):
        a_msg = _segsum_kernel(h, a2b_flat)
        am = _mm(a_msg, W_h[i], relu=False, bm=512)
        hm = _mm(h, W_h[i], relu=False, bm=2000)
        h = _edge_update_kernel(am, hm, inp, b2a, b2revb)

    return _mm(h, W_o, relu=True, bm=2000)
